# Initial kernel scaffold; baseline (speedup 1.0000x reference)
import jax
import jax.numpy as jnp
from jax.experimental import pallas as pl


def kernel(x, edge_index, W, b):
    raise NotImplementedError("write your pallas kernel here")


# speedup vs baseline: 1.5610x; 1.5610x over previous
import functools

import jax
import jax.numpy as jnp
from jax import lax
from jax.experimental import pallas as pl
from jax.experimental.pallas import tpu as pltpu
from jax.experimental.pallas import tpu_sc as plsc

_LANES = 16


def _mm_body(x_ref, w_ref, b_ref, y_ref):
    y_ref[...] = (
        jnp.dot(x_ref[...], w_ref[...], preferred_element_type=jnp.float32)
        + b_ref[...]
    )


def _matmul_bias(x, W, bhalf):
    n, d_in = x.shape
    d_out = W.shape[1]
    bn = 2000
    assert n % bn == 0
    return pl.pallas_call(
        _mm_body,
        grid=(n // bn,),
        in_specs=[
            pl.BlockSpec((bn, d_in), lambda i: (i, 0)),
            pl.BlockSpec((d_in, d_out), lambda i: (0, 0)),
            pl.BlockSpec((1, d_out), lambda i: (0, 0)),
        ],
        out_specs=pl.BlockSpec((bn, d_out), lambda i: (i, 0)),
        out_shape=jax.ShapeDtypeStruct((n, d_out), jnp.float32),
    )(x, W, bhalf)


def _gather_add_sc(y, row, col):
    n, d = y.shape
    e = row.shape[0]
    info = plsc.get_sparse_core_info()
    nw = info.num_cores * info.num_subcores
    assert e % nw == 0
    epw = e // nw
    c = 40
    assert epw % c == 0 and c % 8 == 0 and c <= 128
    n_chunks = epw // c
    mesh = plsc.VectorSubcoreMesh(core_axis_name="c", subcore_axis_name="s")

    @functools.partial(
        pl.kernel,
        mesh=mesh,
        out_type=jax.ShapeDtypeStruct((e, d), jnp.float32),
        scratch_types=[
            pltpu.VMEM((epw,), jnp.int32),
            pltpu.VMEM((epw,), jnp.int32),
            pltpu.VMEM((c, d), jnp.float32),
            pltpu.VMEM((c, d), jnp.float32),
            pltpu.SemaphoreType.DMA,
            pltpu.SemaphoreType.DMA,
        ],
    )
    def k(y_hbm, row_hbm, col_hbm, out_hbm, idx_r, idx_c, buf_a, buf_b,
          sem_a, sem_b):
        wid = lax.axis_index("s") * info.num_cores + lax.axis_index("c")
        base_w = wid * epw
        pltpu.sync_copy(row_hbm.at[pl.ds(base_w, epw)], idx_r)
        pltpu.sync_copy(col_hbm.at[pl.ds(base_w, epw)], idx_c)

        def chunk_body(j, _):
            off = j * c
            cp_a = pltpu.async_copy(
                y_hbm.at[idx_r.at[pl.ds(off, c)]], buf_a, sem_a)
            cp_b = pltpu.async_copy(
                y_hbm.at[idx_c.at[pl.ds(off, c)]], buf_b, sem_b)
            cp_a.wait()
            cp_b.wait()

            def add_row(i, _):
                for jj in range(d // _LANES):
                    sl = pl.ds(jj * _LANES, _LANES)
                    plsc.addupdate(buf_a.at[i, sl], buf_b[i, sl])
                return 0

            lax.fori_loop(0, c, add_row, 0)
            pltpu.sync_copy(buf_a, out_hbm.at[pl.ds(base_w + off, c)])
            return 0

        lax.fori_loop(0, n_chunks, chunk_body, 0)

    return k(y, row, col)


def kernel(x, edge_index, W, b):
    n = x.shape[0]
    row = jnp.clip(edge_index[0].astype(jnp.int32), 0, n - 1)
    col = jnp.clip(edge_index[1].astype(jnp.int32), 0, n - 1)
    bhalf = (0.5 * b).reshape(1, -1).astype(jnp.float32)
    y = _matmul_bias(x, W, bhalf)
    return _gather_add_sc(y, row, col)


# speedup vs baseline: 1.7682x; 1.1327x over previous
import functools

import jax
import jax.numpy as jnp
from jax import lax
from jax.experimental import pallas as pl
from jax.experimental.pallas import tpu as pltpu
from jax.experimental.pallas import tpu_sc as plsc

_LANES = 16


def _mm_body(x_ref, w_ref, b_ref, y_ref):
    y_ref[...] = (
        jnp.dot(x_ref[...], w_ref[...], preferred_element_type=jnp.float32)
        + b_ref[...]
    )


def _matmul_bias(x, W, bhalf):
    n, d_in = x.shape
    d_out = W.shape[1]
    bn = 2000
    assert n % bn == 0
    return pl.pallas_call(
        _mm_body,
        grid=(n // bn,),
        in_specs=[
            pl.BlockSpec((bn, d_in), lambda i: (i, 0)),
            pl.BlockSpec((d_in, d_out), lambda i: (0, 0)),
            pl.BlockSpec((1, d_out), lambda i: (0, 0)),
        ],
        out_specs=pl.BlockSpec((bn, d_out), lambda i: (i, 0)),
        out_shape=jax.ShapeDtypeStruct((n, d_out), jnp.float32),
    )(x, W, bhalf)


_C = 40
_NBUF = 3


def _gather_add_sc(y, idx2, e):
    n, d = y.shape
    info = plsc.get_sparse_core_info()
    nw = info.num_cores * info.num_subcores
    assert e % (nw * _C) == 0
    nch = e // (nw * _C)
    n_outer = (nch + _NBUF - 1) // _NBUF
    mesh = plsc.VectorSubcoreMesh(core_axis_name="c", subcore_axis_name="s")

    @functools.partial(
        pl.kernel,
        mesh=mesh,
        out_type=jax.ShapeDtypeStruct((e, d), jnp.float32),
        scratch_types=(
            [pltpu.VMEM((2 * _C, d), jnp.float32) for _ in range(_NBUF)]
            + [pltpu.VMEM((2 * _C,), jnp.int32) for _ in range(_NBUF)]
            + [pltpu.SemaphoreType.DMA] * (3 * _NBUF)
        ),
    )
    def k(y_hbm, idx2_hbm, out_hbm,
          buf0, buf1, buf2, ib0, ib1, ib2,
          gs0, gs1, gs2, is0, is1, is2, os0, os1, os2):
        bufs = (buf0, buf1, buf2)
        ibufs = (ib0, ib1, ib2)
        gsem = (gs0, gs1, gs2)
        isem = (is0, is1, is2)
        osem = (os0, os1, os2)
        wid = lax.axis_index("s") * info.num_cores + lax.axis_index("c")
        cbase = wid * nch

        def fire_idx(j, b):
            pltpu.async_copy(
                idx2_hbm.at[pl.ds((cbase + j) * 2 * _C, 2 * _C)],
                ibufs[b], isem[b])

        def fire_gather(j, b):
            del j
            pltpu.async_copy(y_hbm.at[ibufs[b]], bufs[b], gsem[b])

        for b in range(_NBUF):
            fire_idx(b, b)
        for b in range(2):
            pltpu.make_async_copy(
                idx2_hbm.at[pl.ds((cbase + b) * 2 * _C, 2 * _C)],
                ibufs[b], isem[b]).wait()
            fire_gather(b, b)

        def slot(j, b):
            b2 = (b + 2) % _NBUF

            @pl.when(j < nch)
            def _process():
                pltpu.make_async_copy(
                    y_hbm.at[ibufs[b]], bufs[b], gsem[b]).wait()

                @pl.when(j + _NBUF < nch)
                def _():
                    fire_idx(j + _NBUF, b)

                def add_row(i, _):
                    for jj in range(d // _LANES):
                        sl = pl.ds(jj * _LANES, _LANES)
                        plsc.addupdate(bufs[b].at[i, sl], bufs[b][_C + i, sl])
                    return 0

                lax.fori_loop(0, _C, add_row, 0)
                pltpu.async_copy(
                    bufs[b].at[pl.ds(0, _C)],
                    out_hbm.at[pl.ds((cbase + j) * _C, _C)], osem[b])

            @pl.when(j + 2 < nch)
            def _next_gather():
                pltpu.make_async_copy(
                    idx2_hbm.at[pl.ds((cbase + j + 2) * 2 * _C, 2 * _C)],
                    ibufs[b2], isem[b2]).wait()

                @pl.when(j >= 1)
                def _():
                    pltpu.make_async_copy(
                        bufs[b2].at[pl.ds(0, _C)],
                        out_hbm.at[pl.ds((cbase + j - 1) * _C, _C)],
                        osem[b2]).wait()

                fire_gather(j + 2, b2)

        def outer(g, _):
            j0 = g * _NBUF
            for b in range(_NBUF):
                slot(j0 + b, b)
            return 0

        lax.fori_loop(0, n_outer, outer, 0)

        for jj in range(nch - _NBUF, nch):
            b = jj % _NBUF
            pltpu.make_async_copy(
                bufs[b].at[pl.ds(0, _C)],
                out_hbm.at[pl.ds((cbase + jj) * _C, _C)], osem[b]).wait()

    return k(y, idx2)


def kernel(x, edge_index, W, b):
    n = x.shape[0]
    e = edge_index.shape[1]
    row = jnp.clip(edge_index[0].astype(jnp.int32), 0, n - 1)
    col = jnp.clip(edge_index[1].astype(jnp.int32), 0, n - 1)
    idx2 = jnp.stack(
        [row.reshape(e // _C, _C), col.reshape(e // _C, _C)], axis=1
    ).reshape(-1)
    bhalf = (0.5 * b).reshape(1, -1).astype(jnp.float32)
    y = _matmul_bias(x, W, bhalf)
    return _gather_add_sc(y, idx2, e)
